# Initial kernel scaffold; baseline (speedup 1.0000x reference)
import jax
import jax.numpy as jnp
from jax.experimental import pallas as pl


def kernel(x, edge_index, W0, b0, W1, b1, W2, b2, gamma1, beta1):
    raise NotImplementedError("write your pallas kernel here")


# speedup vs baseline: 3.3367x; 3.3367x over previous
import functools

import jax
import jax.numpy as jnp
from jax import lax
from jax.experimental import pallas as pl
from jax.experimental.pallas import tpu as pltpu
from jax.experimental.pallas import tpu_sc as plsc

N = 10000
E = 320000
D = 128

NC = 2
NS = 16
NW = NC * NS
E_PAD = 327680
EPT = E_PAD // NW
PKR = EPT // 128
CGA = 64
SUB = 32
NCHUNK = EPT // CGA
JUNK = N
NP = 10112
RPS = NP // NS
NH = 10016
DEGW = 16

_MESH = dict(core_axis_name="c", subcore_axis_name="s",
             num_cores=NC, num_subcores=NS)
_SC_PARAMS = pltpu.CompilerParams(needs_layout_passes=False)


def _count_body(pk3, a3, p_out, pk_v, a_v, sbuf, dring, agg_sh,
                ssem0, ssem1):
    c_ax = lax.axis_index("c")
    s_ax = lax.axis_index("s")
    wid = s_ax * NC + c_ax
    pltpu.sync_copy(pk3.at[wid], pk_v)
    pltpu.sync_copy(a3.at[wid], a_v)

    def zfill(i, _):
        for f in range(D // 16):
            sbuf[0, i, pl.ds(f * 16, 16)] = jnp.zeros((16,), jnp.float32)
        return 0
    lax.fori_loop(0, SUB, zfill, 0, unroll=4)
    nz = RPS // SUB
    for z in range(nz):
        pltpu.sync_copy(sbuf.at[0],
                        agg_sh.at[pl.ds(s_ax * RPS + z * SUB, SUB)])
    rem = RPS - nz * SUB
    pltpu.sync_copy(sbuf.at[0, pl.ds(0, rem)],
                    agg_sh.at[pl.ds(s_ax * RPS + nz * SUB, rem)])
    plsc.subcore_barrier()

    ssems = (ssem0, ssem1)

    def body(k, _):
        for half in range(2):
            sc = 2 * k + half
            _dec_dst(sc, pk_v, dring)
            row = lax.shift_right_logical(sc, 2)

            @pl.when(k >= 1)
            def _():
                pltpu.make_async_copy(
                    sbuf.at[half], agg_sh.at[dring.at[(sc + 2) % 4]],
                    ssems[half]).wait()
            for e in range(SUB):
                cb = (sc & 3) * SUB
                coef = plsc.load_gather(
                    a_v, [jnp.full((16,), row, jnp.int32),
                          jnp.full((16,), cb + e, jnp.int32)])
                for f in range(D // 16):
                    sbuf[half, e, pl.ds(f * 16, 16)] = coef
            pltpu.async_copy(sbuf.at[half], agg_sh.at[dring.at[sc % 4]],
                             ssems[half], add=True)
        return 0
    lax.fori_loop(0, EPT // SUB // 2, body, 0)

    last = EPT // SUB - 2
    for half in range(2):
        pltpu.make_async_copy(sbuf.at[half],
                              agg_sh.at[dring.at[(last + half) % 4]],
                              ssems[half]).wait()
    plsc.subcore_barrier()

    pltpu.sync_copy(agg_sh.at[pl.ds(s_ax * RPS, RPS)],
                    p_out.at[c_ax, pl.ds(s_ax * RPS, RPS)])


def _count(pk3, a3):
    k = pl.kernel(
        _count_body,
        out_type=jax.ShapeDtypeStruct((NC, NP, D), jnp.float32),
        mesh=plsc.VectorSubcoreMesh(**_MESH),
        scratch_types=[
            pltpu.VMEM((PKR, 128), jnp.int32),
            pltpu.VMEM((PKR, 128), jnp.float32),
            pltpu.VMEM((2, SUB, D), jnp.float32),
            pltpu.VMEM((4, SUB), jnp.int32),
            pltpu.VMEM_SHARED((NP, D), jnp.float32),
            pltpu.SemaphoreType.DMA,
            pltpu.SemaphoreType.DMA,
        ],
        compiler_params=_SC_PARAMS,
        name="sc_count",
    )
    return k(pk3, a3)


def _dec_src(j, pk_v, sring):
    row = lax.shift_right_logical(j, 1)
    cb = (j & 1) * CGA
    for k in range(CGA // 16):
        v = pk_v[row, pl.ds(cb + k * 16, 16)]
        sring[j % 4, pl.ds(k * 16, 16)] = lax.shift_right_logical(v, 14)


def _dec_dst(s, pk_v, dring):
    row = lax.shift_right_logical(s, 2)
    cb = (s & 3) * SUB
    for k in range(SUB // 16):
        v = pk_v[row, pl.ds(cb + k * 16, 16)]
        dring[s % 4, pl.ds(k * 16, 16)] = v & 16383


def _spmm_body(h, pk3, a3, p_out, pk_v, a_v, gbuf, sbuf, sring, dring,
               agg_sh, gsem0, gsem1, ssem0, ssem1):
    c_ax = lax.axis_index("c")
    s_ax = lax.axis_index("s")
    wid = s_ax * NC + c_ax
    pltpu.sync_copy(pk3.at[wid], pk_v)
    pltpu.sync_copy(a3.at[wid], a_v)

    def zfill(i, _):
        for f in range(D // 16):
            gbuf[0, i, pl.ds(f * 16, 16)] = jnp.zeros((16,), jnp.float32)
        return 0
    lax.fori_loop(0, CGA, zfill, 0, unroll=4)
    for z in range(9):
        pltpu.sync_copy(gbuf.at[0], agg_sh.at[pl.ds(s_ax * RPS + z * CGA, CGA)])
    rem = RPS - 9 * CGA
    pltpu.sync_copy(gbuf.at[0, pl.ds(0, rem)],
                    agg_sh.at[pl.ds(s_ax * RPS + 9 * CGA, rem)])

    _dec_src(jnp.int32(0), pk_v, sring)
    _dec_src(jnp.int32(1), pk_v, sring)
    pltpu.async_copy(h.at[sring.at[0]], gbuf.at[0], gsem0)
    pltpu.async_copy(h.at[sring.at[1]], gbuf.at[1], gsem1)
    plsc.subcore_barrier()

    gsems = (gsem0, gsem1)
    ssems = (ssem0, ssem1)

    def one_chunk(j, b, first):
        pltpu.make_async_copy(h.at[sring.at[j % 4]], gbuf.at[b],
                              gsems[b]).wait()
        _dec_dst(2 * j, pk_v, dring)
        _dec_dst(2 * j + 1, pk_v, dring)
        row = lax.shift_right_logical(j, 1)
        cb = (j & 1) * CGA
        for half in range(2):
            def _wait_prev():
                pltpu.make_async_copy(
                    sbuf.at[half], agg_sh.at[dring.at[(2 * j + half + 2) % 4]],
                    ssems[half]).wait()
            if first is None:
                _wait_prev()
            else:
                pl.when(jnp.logical_not(first))(_wait_prev)
            for e in range(SUB):
                ge = half * SUB + e
                coef = plsc.load_gather(
                    a_v, [jnp.full((16,), row, jnp.int32),
                          jnp.full((16,), cb + ge, jnp.int32)])
                for f in range(D // 16):
                    sl = pl.ds(f * 16, 16)
                    sbuf[half, e, sl] = gbuf[b, ge, sl] * coef
            pltpu.async_copy(sbuf.at[half],
                             agg_sh.at[dring.at[(2 * j + half) % 4]],
                             ssems[half], add=True)

        @pl.when(j < NCHUNK - 2)
        def _():
            _dec_src(j + 2, pk_v, sring)
            pltpu.async_copy(h.at[sring.at[(j + 2) % 4]], gbuf.at[b],
                             gsems[b])

    def body(k, _):
        one_chunk(2 * k, 0, k < 1)
        one_chunk(2 * k + 1, 1, None)
        return 0
    lax.fori_loop(0, NCHUNK // 2, body, 0)

    for half in range(2):
        pltpu.make_async_copy(sbuf.at[half],
                              agg_sh.at[dring.at[(318 + half) % 4]],
                              ssems[half]).wait()
    plsc.subcore_barrier()

    pltpu.sync_copy(agg_sh.at[pl.ds(s_ax * RPS, RPS)],
                    p_out.at[c_ax, pl.ds(s_ax * RPS, RPS)])


def _spmm(h, pk3, a3):
    k = pl.kernel(
        _spmm_body,
        out_type=jax.ShapeDtypeStruct((NC, NP, D), jnp.float32),
        mesh=plsc.VectorSubcoreMesh(**_MESH),
        scratch_types=[
            pltpu.VMEM((PKR, 128), jnp.int32),
            pltpu.VMEM((PKR, 128), jnp.float32),
            pltpu.VMEM((2, CGA, D), jnp.float32),
            pltpu.VMEM((2, SUB, D), jnp.float32),
            pltpu.VMEM((4, CGA), jnp.int32),
            pltpu.VMEM((4, SUB), jnp.int32),
            pltpu.VMEM_SHARED((NP, D), jnp.float32),
            pltpu.SemaphoreType.DMA,
            pltpu.SemaphoreType.DMA,
            pltpu.SemaphoreType.DMA,
            pltpu.SemaphoreType.DMA,
        ],
        compiler_params=_SC_PARAMS,
        name="sc_spmm",
    )
    return k(h, pk3, a3)


def _norms_body(outp_ref, inp_ref, x_ref, norms_ref, xs_ref):
    outd = outp_ref[0, :, 0:DEGW] + outp_ref[1, :, 0:DEGW]
    ind = inp_ref[0, :, 0:DEGW] + inp_ref[1, :, 0:DEGW]
    ns = jnp.where(outd > 0, lax.rsqrt(jnp.maximum(outd, 1.0)), 0.0)
    nd = jnp.where(ind > 0, lax.rsqrt(jnp.maximum(ind, 1.0)), 0.0)
    norms_ref[0] = ns
    norms_ref[1] = nd
    xs = x_ref[...] * ns[:N, 0:1]
    xs_ref[...] = jnp.concatenate(
        [xs, jnp.zeros((NH - N, D), jnp.float32)], axis=0)


def _norms_prescale(outp, inp, x):
    return pl.pallas_call(
        _norms_body,
        out_shape=(
            jax.ShapeDtypeStruct((2, NP, DEGW), jnp.float32),
            jax.ShapeDtypeStruct((NH, D), jnp.float32),
        ),
    )(outp, inp, x)


def _dense_body(p_ref, norms_ref, w_ref, b_ref, g_ref, bt_ref, o_ref, *,
                act, bn, prescale):
    nd = norms_ref[1, :N, 0:1]
    agg = (p_ref[0, :N] + p_ref[1, :N]) * nd
    y = jnp.dot(agg, w_ref[...], preferred_element_type=jnp.float32)
    y = y + b_ref[...][0:1, :]
    if act:
        y = jnp.maximum(y, 0.0)
    if bn:
        mu = jnp.mean(y, axis=0, keepdims=True)
        var = jnp.mean((y - mu) ** 2, axis=0, keepdims=True)
        y = (y - mu) * lax.rsqrt(var + 1e-5) * g_ref[...][0:1, :] \
            + bt_ref[...][0:1, :]
    if prescale:
        y = y * norms_ref[0, :N, 0:1]
        y = jnp.concatenate([y, jnp.zeros((NH - N, D), jnp.float32)], axis=0)
    o_ref[...] = y


def _dense_stage(p, norms, w, b, gamma, beta, act, bn, prescale):
    body = functools.partial(_dense_body, act=act, bn=bn, prescale=prescale)
    rows = NH if prescale else N
    return pl.pallas_call(
        body,
        out_shape=jax.ShapeDtypeStruct((rows, D), jnp.float32),
    )(p, norms, w, b.reshape(1, D), gamma.reshape(1, D), beta.reshape(1, D))


def kernel(x, edge_index, W0, b0, W1, b1, W2, b2, gamma1, beta1):
    pad = jnp.full((E_PAD - E,), JUNK, jnp.int32)
    srcp = jnp.concatenate([edge_index[0], pad])
    dstp = jnp.concatenate([edge_index[1], pad])
    pk3 = ((srcp << 14) | dstp).reshape(NW, PKR, 128)
    nk = jax.random.key(42)
    apad = jnp.zeros((E_PAD - E,), jnp.float32)

    def mk_a(i):
        a = 1.0 + jax.random.normal(jax.random.fold_in(nk, i), (E,),
                                    jnp.float32)
        return jnp.concatenate([a, apad]).reshape(NW, PKR, 128)

    ones3 = jnp.concatenate(
        [jnp.ones((E,), jnp.float32), apad]).reshape(NW, PKR, 128)
    pk3s = ((dstp << 14) | srcp).reshape(NW, PKR, 128)
    outp = _count(pk3s, ones3)
    inp = _count(pk3, ones3)
    norms, xs = _norms_prescale(outp, inp, x)

    p = _spmm(xs, pk3, mk_a(0))
    h = _dense_stage(p, norms, W0, b0, gamma1, beta1, True, False, True)
    p = _spmm(h, pk3, mk_a(1))
    h = _dense_stage(p, norms, W1, b1, gamma1, beta1, True, True, True)
    p = _spmm(h, pk3, mk_a(2))
    out = _dense_stage(p, norms, W2, b2, gamma1, beta1, False, False, False)
    return out


# speedup vs baseline: 3.4906x; 1.0461x over previous
import functools

import jax
import jax.numpy as jnp
from jax import lax
from jax.experimental import pallas as pl
from jax.experimental.pallas import tpu as pltpu
from jax.experimental.pallas import tpu_sc as plsc

N = 10000
E = 320000
D = 128

NC = 2
NS = 16
NW = NC * NS
E_PAD = 327680
EPT = E_PAD // NW
PKR = EPT // 128
CGA = 64
SUB = 32
NCHUNK = EPT // CGA
JUNK = N
NP = 10112
RPS = NP // NS
NH = 10016
DEGW = 16

_MESH = dict(core_axis_name="c", subcore_axis_name="s",
             num_cores=NC, num_subcores=NS)
_SC_PARAMS = pltpu.CompilerParams(needs_layout_passes=False)


def _count_body(pk3, p_out, pk_v, idx_v, sbuf, agg_sh, ssem, *, use_src):
    c_ax = lax.axis_index("c")
    s_ax = lax.axis_index("s")
    wid = s_ax * NC + c_ax
    pltpu.sync_copy(pk3.at[wid], pk_v)

    def dec(r, _):
        for k in range(8):
            sl = pl.ds(k * 16, 16)
            v = pk_v[r, sl]
            if use_src:
                idx_v[r, sl] = lax.shift_right_logical(v, 14)
            else:
                idx_v[r, sl] = v & 16383
        return 0
    lax.fori_loop(0, PKR, dec, 0, unroll=2)

    def zfill(i, _):
        for f in range(D // 16):
            sbuf[i, pl.ds(f * 16, 16)] = jnp.zeros((16,), jnp.float32)
        return 0
    lax.fori_loop(0, 128, zfill, 0, unroll=4)

    def fill(i, _):
        for f in range(D // 16):
            sbuf[i, pl.ds(f * 16, 16)] = jnp.full((16,), 1.0, jnp.float32)
        return 0

    zrows = 128
    for z in range(RPS // zrows):
        pltpu.sync_copy(sbuf, agg_sh.at[pl.ds(s_ax * RPS + z * zrows, zrows)])
    rem = RPS - (RPS // zrows) * zrows
    pltpu.sync_copy(sbuf.at[pl.ds(0, rem)],
                    agg_sh.at[pl.ds(s_ax * RPS + (RPS // zrows) * zrows, rem)])
    lax.fori_loop(0, 128, fill, 0, unroll=4)
    plsc.subcore_barrier()

    def fire(j, _):
        pltpu.async_copy(sbuf, agg_sh.at[idx_v.at[j]], ssem, add=True)
        return 0
    lax.fori_loop(0, PKR, fire, 0)

    def drain(j, _):
        pltpu.make_async_copy(sbuf, agg_sh.at[idx_v.at[0]], ssem).wait()
        return 0
    lax.fori_loop(0, PKR, drain, 0)
    plsc.subcore_barrier()

    pltpu.sync_copy(agg_sh.at[pl.ds(s_ax * RPS, RPS)],
                    p_out.at[c_ax, pl.ds(s_ax * RPS, RPS)])


def _count(pk3, use_src):
    k = pl.kernel(
        functools.partial(_count_body, use_src=use_src),
        out_type=jax.ShapeDtypeStruct((NC, NP, D), jnp.float32),
        mesh=plsc.VectorSubcoreMesh(**_MESH),
        scratch_types=[
            pltpu.VMEM((PKR, 128), jnp.int32),
            pltpu.VMEM((PKR, 128), jnp.int32),
            pltpu.VMEM((128, D), jnp.float32),
            pltpu.VMEM_SHARED((NP, D), jnp.float32),
            pltpu.SemaphoreType.DMA,
        ],
        compiler_params=_SC_PARAMS,
        name="sc_count_src" if use_src else "sc_count_dst",
    )
    return k(pk3)


def _dec_src(j, pk_v, sring):
    row = lax.shift_right_logical(j, 1)
    cb = (j & 1) * CGA
    for k in range(CGA // 16):
        v = pk_v[row, pl.ds(cb + k * 16, 16)]
        sring[j % 4, pl.ds(k * 16, 16)] = lax.shift_right_logical(v, 14)


def _dec_dst(s, pk_v, dring):
    row = lax.shift_right_logical(s, 2)
    cb = (s & 3) * SUB
    for k in range(SUB // 16):
        v = pk_v[row, pl.ds(cb + k * 16, 16)]
        dring[s % 4, pl.ds(k * 16, 16)] = v & 16383


def _spmm_body(h, pk3, a3, p_out, pk_v, a_v, gbuf, sbuf, sring, dring,
               agg_sh, gsem0, gsem1, ssem0, ssem1):
    c_ax = lax.axis_index("c")
    s_ax = lax.axis_index("s")
    wid = s_ax * NC + c_ax
    pltpu.sync_copy(pk3.at[wid], pk_v)
    pltpu.sync_copy(a3.at[wid], a_v)

    def zfill(i, _):
        for f in range(D // 16):
            gbuf[0, i, pl.ds(f * 16, 16)] = jnp.zeros((16,), jnp.float32)
        return 0
    lax.fori_loop(0, CGA, zfill, 0, unroll=4)
    for z in range(9):
        pltpu.sync_copy(gbuf.at[0], agg_sh.at[pl.ds(s_ax * RPS + z * CGA, CGA)])
    rem = RPS - 9 * CGA
    pltpu.sync_copy(gbuf.at[0, pl.ds(0, rem)],
                    agg_sh.at[pl.ds(s_ax * RPS + 9 * CGA, rem)])

    _dec_src(jnp.int32(0), pk_v, sring)
    _dec_src(jnp.int32(1), pk_v, sring)
    pltpu.async_copy(h.at[sring.at[0]], gbuf.at[0], gsem0)
    pltpu.async_copy(h.at[sring.at[1]], gbuf.at[1], gsem1)
    plsc.subcore_barrier()

    gsems = (gsem0, gsem1)
    ssems = (ssem0, ssem1)

    def one_chunk(j, b, first):
        pltpu.make_async_copy(h.at[sring.at[j % 4]], gbuf.at[b],
                              gsems[b]).wait()
        _dec_dst(2 * j, pk_v, dring)
        _dec_dst(2 * j + 1, pk_v, dring)
        row = lax.shift_right_logical(j, 1)
        cb = (j & 1) * CGA
        for half in range(2):
            def _wait_prev():
                pltpu.make_async_copy(
                    sbuf.at[half], agg_sh.at[dring.at[(2 * j + half + 2) % 4]],
                    ssems[half]).wait()
            if first is None:
                _wait_prev()
            else:
                pl.when(jnp.logical_not(first))(_wait_prev)
            coefs = [
                plsc.load_gather(
                    a_v, [jnp.full((16,), row, jnp.int32),
                          jnp.full((16,), cb + half * SUB + e, jnp.int32)])
                for e in range(SUB)]
            for f in range(D // 16):
                sl = pl.ds(f * 16, 16)
                for e in range(SUB):
                    sbuf[half, e, sl] = gbuf[b, half * SUB + e, sl] * coefs[e]
            pltpu.async_copy(sbuf.at[half],
                             agg_sh.at[dring.at[(2 * j + half) % 4]],
                             ssems[half], add=True)

        @pl.when(j < NCHUNK - 2)
        def _():
            _dec_src(j + 2, pk_v, sring)
            pltpu.async_copy(h.at[sring.at[(j + 2) % 4]], gbuf.at[b],
                             gsems[b])

    def body(k, _):
        one_chunk(2 * k, 0, k < 1)
        one_chunk(2 * k + 1, 1, None)
        return 0
    lax.fori_loop(0, NCHUNK // 2, body, 0)

    for half in range(2):
        pltpu.make_async_copy(sbuf.at[half],
                              agg_sh.at[dring.at[(318 + half) % 4]],
                              ssems[half]).wait()
    plsc.subcore_barrier()

    pltpu.sync_copy(agg_sh.at[pl.ds(s_ax * RPS, RPS)],
                    p_out.at[c_ax, pl.ds(s_ax * RPS, RPS)])


def _spmm(h, pk3, a3):
    k = pl.kernel(
        _spmm_body,
        out_type=jax.ShapeDtypeStruct((NC, NP, D), jnp.float32),
        mesh=plsc.VectorSubcoreMesh(**_MESH),
        scratch_types=[
            pltpu.VMEM((PKR, 128), jnp.int32),
            pltpu.VMEM((PKR, 128), jnp.float32),
            pltpu.VMEM((2, CGA, D), jnp.float32),
            pltpu.VMEM((2, SUB, D), jnp.float32),
            pltpu.VMEM((4, CGA), jnp.int32),
            pltpu.VMEM((4, SUB), jnp.int32),
            pltpu.VMEM_SHARED((NP, D), jnp.float32),
            pltpu.SemaphoreType.DMA,
            pltpu.SemaphoreType.DMA,
            pltpu.SemaphoreType.DMA,
            pltpu.SemaphoreType.DMA,
        ],
        compiler_params=_SC_PARAMS,
        name="sc_spmm",
    )
    return k(h, pk3, a3)


def _norms_body(outp_ref, inp_ref, x_ref, norms_ref, xs_ref):
    outd = outp_ref[0, :, 0:DEGW] + outp_ref[1, :, 0:DEGW]
    ind = inp_ref[0, :, 0:DEGW] + inp_ref[1, :, 0:DEGW]
    ns = jnp.where(outd > 0, lax.rsqrt(jnp.maximum(outd, 1.0)), 0.0)
    nd = jnp.where(ind > 0, lax.rsqrt(jnp.maximum(ind, 1.0)), 0.0)
    norms_ref[0] = ns
    norms_ref[1] = nd
    xs = x_ref[...] * ns[:N, 0:1]
    xs_ref[...] = jnp.concatenate(
        [xs, jnp.zeros((NH - N, D), jnp.float32)], axis=0)


def _norms_prescale(outp, inp, x):
    return pl.pallas_call(
        _norms_body,
        out_shape=(
            jax.ShapeDtypeStruct((2, NP, DEGW), jnp.float32),
            jax.ShapeDtypeStruct((NH, D), jnp.float32),
        ),
    )(outp, inp, x)


def _dense_body(p_ref, norms_ref, w_ref, b_ref, g_ref, bt_ref, o_ref, *,
                act, bn, prescale):
    nd = norms_ref[1, :N, 0:1]
    agg = (p_ref[0, :N] + p_ref[1, :N]) * nd
    y = jnp.dot(agg, w_ref[...], preferred_element_type=jnp.float32)
    y = y + b_ref[...][0:1, :]
    if act:
        y = jnp.maximum(y, 0.0)
    if bn:
        mu = jnp.mean(y, axis=0, keepdims=True)
        var = jnp.mean((y - mu) ** 2, axis=0, keepdims=True)
        y = (y - mu) * lax.rsqrt(var + 1e-5) * g_ref[...][0:1, :] \
            + bt_ref[...][0:1, :]
    if prescale:
        y = y * norms_ref[0, :N, 0:1]
        y = jnp.concatenate([y, jnp.zeros((NH - N, D), jnp.float32)], axis=0)
    o_ref[...] = y


def _dense_stage(p, norms, w, b, gamma, beta, act, bn, prescale):
    body = functools.partial(_dense_body, act=act, bn=bn, prescale=prescale)
    rows = NH if prescale else N
    return pl.pallas_call(
        body,
        out_shape=jax.ShapeDtypeStruct((rows, D), jnp.float32),
    )(p, norms, w, b.reshape(1, D), gamma.reshape(1, D), beta.reshape(1, D))


def kernel(x, edge_index, W0, b0, W1, b1, W2, b2, gamma1, beta1):
    pad = jnp.full((E_PAD - E,), JUNK, jnp.int32)
    srcp = jnp.concatenate([edge_index[0], pad])
    dstp = jnp.concatenate([edge_index[1], pad])
    pk3 = ((srcp << 14) | dstp).reshape(NW, PKR, 128)
    nk = jax.random.key(42)
    apad = jnp.zeros((E_PAD - E,), jnp.float32)

    def mk_a(i):
        a = 1.0 + jax.random.normal(jax.random.fold_in(nk, i), (E,),
                                    jnp.float32)
        return jnp.concatenate([a, apad]).reshape(NW, PKR, 128)

    outp = _count(pk3, True)
    inp = _count(pk3, False)
    norms, xs = _norms_prescale(outp, inp, x)

    p = _spmm(xs, pk3, mk_a(0))
    h = _dense_stage(p, norms, W0, b0, gamma1, beta1, True, False, True)
    p = _spmm(h, pk3, mk_a(1))
    h = _dense_stage(p, norms, W1, b1, gamma1, beta1, True, True, True)
    p = _spmm(h, pk3, mk_a(2))
    out = _dense_stage(p, norms, W2, b2, gamma1, beta1, False, False, False)
    return out


# speedup vs baseline: 3.5046x; 1.0040x over previous
import functools

import jax
import jax.numpy as jnp
from jax import lax
from jax.experimental import pallas as pl
from jax.experimental.pallas import tpu as pltpu
from jax.experimental.pallas import tpu_sc as plsc

N = 10000
E = 320000
D = 128

NC = 2
NS = 16
NW = NC * NS
E_PAD = 327680
EPT = E_PAD // NW
PKR = EPT // 128
CGA = 32
SUB = 32
NCHUNK = EPT // CGA
JUNK = N
NP = 10112
RPS = NP // NS
NH = 10016
DEGW = 16

_MESH = dict(core_axis_name="c", subcore_axis_name="s",
             num_cores=NC, num_subcores=NS)
_SC_PARAMS = pltpu.CompilerParams(needs_layout_passes=False)


def _count_body(pk3, p_out, pk_v, idx_v, sbuf, agg_sh, ssem, *, use_src):
    c_ax = lax.axis_index("c")
    s_ax = lax.axis_index("s")
    wid = s_ax * NC + c_ax
    pltpu.sync_copy(pk3.at[wid], pk_v)

    def dec(r, _):
        for k in range(8):
            sl = pl.ds(k * 16, 16)
            v = pk_v[r, sl]
            if use_src:
                idx_v[r, sl] = lax.shift_right_logical(v, 14)
            else:
                idx_v[r, sl] = v & 16383
        return 0
    lax.fori_loop(0, PKR, dec, 0, unroll=2)

    def zfill(i, _):
        for f in range(D // 16):
            sbuf[i, pl.ds(f * 16, 16)] = jnp.zeros((16,), jnp.float32)
        return 0
    lax.fori_loop(0, 128, zfill, 0, unroll=4)

    def fill(i, _):
        for f in range(D // 16):
            sbuf[i, pl.ds(f * 16, 16)] = jnp.full((16,), 1.0, jnp.float32)
        return 0

    zrows = 128
    for z in range(RPS // zrows):
        pltpu.sync_copy(sbuf, agg_sh.at[pl.ds(s_ax * RPS + z * zrows, zrows)])
    rem = RPS - (RPS // zrows) * zrows
    pltpu.sync_copy(sbuf.at[pl.ds(0, rem)],
                    agg_sh.at[pl.ds(s_ax * RPS + (RPS // zrows) * zrows, rem)])
    lax.fori_loop(0, 128, fill, 0, unroll=4)
    plsc.subcore_barrier()

    def fire(j, _):
        pltpu.async_copy(sbuf, agg_sh.at[idx_v.at[j]], ssem, add=True)
        return 0
    lax.fori_loop(0, PKR, fire, 0)

    def drain(j, _):
        pltpu.make_async_copy(sbuf, agg_sh.at[idx_v.at[0]], ssem).wait()
        return 0
    lax.fori_loop(0, PKR, drain, 0)
    plsc.subcore_barrier()

    pltpu.sync_copy(agg_sh.at[pl.ds(s_ax * RPS, RPS)],
                    p_out.at[c_ax, pl.ds(s_ax * RPS, RPS)])


def _count(pk3, use_src):
    k = pl.kernel(
        functools.partial(_count_body, use_src=use_src),
        out_type=jax.ShapeDtypeStruct((NC, NP, D), jnp.float32),
        mesh=plsc.VectorSubcoreMesh(**_MESH),
        scratch_types=[
            pltpu.VMEM((PKR, 128), jnp.int32),
            pltpu.VMEM((PKR, 128), jnp.int32),
            pltpu.VMEM((128, D), jnp.float32),
            pltpu.VMEM_SHARED((NP, D), jnp.float32),
            pltpu.SemaphoreType.DMA,
        ],
        compiler_params=_SC_PARAMS,
        name="sc_count_src" if use_src else "sc_count_dst",
    )
    return k(pk3)


def _dec_src(j, pk_v, sring):
    row = lax.shift_right_logical(j, 2)
    cb = (j & 3) * CGA
    for k in range(CGA // 16):
        v = pk_v[row, pl.ds(cb + k * 16, 16)]
        sring[j % 8, pl.ds(k * 16, 16)] = lax.shift_right_logical(v, 14)


def _dec_dst(j, pk_v, dring):
    row = lax.shift_right_logical(j, 2)
    cb = (j & 3) * SUB
    for k in range(SUB // 16):
        v = pk_v[row, pl.ds(cb + k * 16, 16)]
        dring[j % 4, pl.ds(k * 16, 16)] = v & 16383


def _spmm_body(h, pk3, a3, p_out, pk_v, a_v, gbuf, sbuf, sring, dring,
               agg_sh, gsem0, gsem1, gsem2, gsem3, ssem0, ssem1):
    c_ax = lax.axis_index("c")
    s_ax = lax.axis_index("s")
    wid = s_ax * NC + c_ax
    pltpu.sync_copy(pk3.at[wid], pk_v)
    pltpu.sync_copy(a3.at[wid], a_v)

    def zfill(i, _):
        for f in range(D // 16):
            gbuf[0, i, pl.ds(f * 16, 16)] = jnp.zeros((16,), jnp.float32)
        return 0
    lax.fori_loop(0, CGA, zfill, 0, unroll=4)
    for z in range(RPS // CGA):
        pltpu.sync_copy(gbuf.at[0],
                        agg_sh.at[pl.ds(s_ax * RPS + z * CGA, CGA)])
    rem = RPS - (RPS // CGA) * CGA
    pltpu.sync_copy(gbuf.at[0, pl.ds(0, rem)],
                    agg_sh.at[pl.ds(s_ax * RPS + (RPS // CGA) * CGA, rem)])

    for b0 in range(4):
        _dec_src(jnp.int32(b0), pk_v, sring)
    gsems_pre = (gsem0, gsem1, gsem2, gsem3)
    for b0 in range(4):
        pltpu.async_copy(h.at[sring.at[b0]], gbuf.at[b0], gsems_pre[b0])
    plsc.subcore_barrier()

    gsems = (gsem0, gsem1, gsem2, gsem3)
    ssems = (ssem0, ssem1)

    def one_chunk(j, b, k):
        sb = b % 2
        pltpu.make_async_copy(h.at[sring.at[j % 8]], gbuf.at[b],
                              gsems[b]).wait()
        _dec_dst(j, pk_v, dring)
        row = lax.shift_right_logical(j, 2)
        cb = (j & 3) * CGA

        def _wait_prev():
            pltpu.make_async_copy(
                sbuf.at[sb], agg_sh.at[dring.at[(j + 2) % 4]],
                ssems[sb]).wait()
        if b >= 2:
            _wait_prev()
        else:
            pl.when(k >= 1)(_wait_prev)
        coefs = [
            plsc.load_gather(
                a_v, [jnp.full((16,), row, jnp.int32),
                      jnp.full((16,), cb + e, jnp.int32)])
            for e in range(CGA)]
        for f in range(D // 16):
            sl = pl.ds(f * 16, 16)
            for e in range(CGA):
                sbuf[sb, e, sl] = gbuf[b, e, sl] * coefs[e]
        pltpu.async_copy(sbuf.at[sb], agg_sh.at[dring.at[j % 4]],
                         ssems[sb], add=True)

        @pl.when(k < NCHUNK // 4 - 1)
        def _():
            _dec_src(j + 4, pk_v, sring)
            pltpu.async_copy(h.at[sring.at[(j + 4) % 8]], gbuf.at[b],
                             gsems[b])

    def body(k, _):
        for b in range(4):
            one_chunk(4 * k + b, b, k)
        return 0
    lax.fori_loop(0, NCHUNK // 4, body, 0)

    for sb in range(2):
        pltpu.make_async_copy(sbuf.at[sb],
                              agg_sh.at[dring.at[(NCHUNK - 2 + sb) % 4]],
                              ssems[sb]).wait()
    plsc.subcore_barrier()

    pltpu.sync_copy(agg_sh.at[pl.ds(s_ax * RPS, RPS)],
                    p_out.at[c_ax, pl.ds(s_ax * RPS, RPS)])


def _spmm(h, pk3, a3):
    k = pl.kernel(
        _spmm_body,
        out_type=jax.ShapeDtypeStruct((NC, NP, D), jnp.float32),
        mesh=plsc.VectorSubcoreMesh(**_MESH),
        scratch_types=[
            pltpu.VMEM((PKR, 128), jnp.int32),
            pltpu.VMEM((PKR, 128), jnp.float32),
            pltpu.VMEM((4, CGA, D), jnp.float32),
            pltpu.VMEM((2, SUB, D), jnp.float32),
            pltpu.VMEM((8, CGA), jnp.int32),
            pltpu.VMEM((4, SUB), jnp.int32),
            pltpu.VMEM_SHARED((NP, D), jnp.float32),
            pltpu.SemaphoreType.DMA,
            pltpu.SemaphoreType.DMA,
            pltpu.SemaphoreType.DMA,
            pltpu.SemaphoreType.DMA,
            pltpu.SemaphoreType.DMA,
            pltpu.SemaphoreType.DMA,
        ],
        compiler_params=_SC_PARAMS,
        name="sc_spmm",
    )
    return k(h, pk3, a3)


def _norms_body(outp_ref, inp_ref, x_ref, norms_ref, xs_ref):
    outd = outp_ref[0, :, 0:DEGW] + outp_ref[1, :, 0:DEGW]
    ind = inp_ref[0, :, 0:DEGW] + inp_ref[1, :, 0:DEGW]
    ns = jnp.where(outd > 0, lax.rsqrt(jnp.maximum(outd, 1.0)), 0.0)
    nd = jnp.where(ind > 0, lax.rsqrt(jnp.maximum(ind, 1.0)), 0.0)
    norms_ref[0] = ns
    norms_ref[1] = nd
    xs = x_ref[...] * ns[:N, 0:1]
    xs_ref[...] = jnp.concatenate(
        [xs, jnp.zeros((NH - N, D), jnp.float32)], axis=0)


def _norms_prescale(outp, inp, x):
    return pl.pallas_call(
        _norms_body,
        out_shape=(
            jax.ShapeDtypeStruct((2, NP, DEGW), jnp.float32),
            jax.ShapeDtypeStruct((NH, D), jnp.float32),
        ),
    )(outp, inp, x)


def _dense_body(p_ref, norms_ref, w_ref, b_ref, g_ref, bt_ref, o_ref, *,
                act, bn, prescale):
    nd = norms_ref[1, :N, 0:1]
    agg = (p_ref[0, :N] + p_ref[1, :N]) * nd
    y = jnp.dot(agg, w_ref[...], preferred_element_type=jnp.float32)
    y = y + b_ref[...][0:1, :]
    if act:
        y = jnp.maximum(y, 0.0)
    if bn:
        mu = jnp.mean(y, axis=0, keepdims=True)
        var = jnp.mean((y - mu) ** 2, axis=0, keepdims=True)
        y = (y - mu) * lax.rsqrt(var + 1e-5) * g_ref[...][0:1, :] \
            + bt_ref[...][0:1, :]
    if prescale:
        y = y * norms_ref[0, :N, 0:1]
        y = jnp.concatenate([y, jnp.zeros((NH - N, D), jnp.float32)], axis=0)
    o_ref[...] = y


def _dense_stage(p, norms, w, b, gamma, beta, act, bn, prescale):
    body = functools.partial(_dense_body, act=act, bn=bn, prescale=prescale)
    rows = NH if prescale else N
    return pl.pallas_call(
        body,
        out_shape=jax.ShapeDtypeStruct((rows, D), jnp.float32),
    )(p, norms, w, b.reshape(1, D), gamma.reshape(1, D), beta.reshape(1, D))


def kernel(x, edge_index, W0, b0, W1, b1, W2, b2, gamma1, beta1):
    pad = jnp.full((E_PAD - E,), JUNK, jnp.int32)
    srcp = jnp.concatenate([edge_index[0], pad])
    dstp = jnp.concatenate([edge_index[1], pad])
    pk3 = ((srcp << 14) | dstp).reshape(NW, PKR, 128)
    nk = jax.random.key(42)
    apad = jnp.zeros((E_PAD - E,), jnp.float32)

    def mk_a(i):
        a = 1.0 + jax.random.normal(jax.random.fold_in(nk, i), (E,),
                                    jnp.float32)
        return jnp.concatenate([a, apad]).reshape(NW, PKR, 128)

    outp = _count(pk3, True)
    inp = _count(pk3, False)
    norms, xs = _norms_prescale(outp, inp, x)

    p = _spmm(xs, pk3, mk_a(0))
    h = _dense_stage(p, norms, W0, b0, gamma1, beta1, True, False, True)
    p = _spmm(h, pk3, mk_a(1))
    h = _dense_stage(p, norms, W1, b1, gamma1, beta1, True, True, True)
    p = _spmm(h, pk3, mk_a(2))
    out = _dense_stage(p, norms, W2, b2, gamma1, beta1, False, False, False)
    return out


# speedup vs baseline: 8.8016x; 2.5114x over previous
import functools

import jax
import jax.numpy as jnp
from jax import lax
from jax.experimental import pallas as pl
from jax.experimental.pallas import tpu as pltpu
from jax.experimental.pallas import tpu_sc as plsc

N = 10000
E = 320000
D = 128

NC = 2
NS = 16
NW = NC * NS
E_PAD = 327680
EPT = E_PAD // NW
PKR = EPT // 128
CGA = 32
SUB = 32
NCHUNK = EPT // CGA
JUNK = N
NP = 10112
RPS = NP // NS
NH = 10016
DEGW = 16

_MESH = dict(core_axis_name="c", subcore_axis_name="s",
             num_cores=NC, num_subcores=NS)
_SC_PARAMS = pltpu.CompilerParams(needs_layout_passes=False)


def _count_body(pk3, p_out, pk_v, idx_v, sbuf, agg_sh, ssem, *, use_src):
    c_ax = lax.axis_index("c")
    s_ax = lax.axis_index("s")
    wid = s_ax * NC + c_ax
    pltpu.sync_copy(pk3.at[wid], pk_v)

    def dec(r, _):
        for k in range(8):
            sl = pl.ds(k * 16, 16)
            v = pk_v[r, sl]
            if use_src:
                idx_v[r, sl] = lax.shift_right_logical(v, 14)
            else:
                idx_v[r, sl] = v & 16383
        return 0
    lax.fori_loop(0, PKR, dec, 0, unroll=2)

    def zfill(i, _):
        for f in range(D // 16):
            sbuf[i, pl.ds(f * 16, 16)] = jnp.zeros((16,), jnp.float32)
        return 0
    lax.fori_loop(0, 128, zfill, 0, unroll=4)

    def fill(i, _):
        for f in range(D // 16):
            sbuf[i, pl.ds(f * 16, 16)] = jnp.full((16,), 1.0, jnp.float32)
        return 0

    zrows = 128
    for z in range(RPS // zrows):
        pltpu.sync_copy(sbuf, agg_sh.at[pl.ds(s_ax * RPS + z * zrows, zrows)])
    rem = RPS - (RPS // zrows) * zrows
    pltpu.sync_copy(sbuf.at[pl.ds(0, rem)],
                    agg_sh.at[pl.ds(s_ax * RPS + (RPS // zrows) * zrows, rem)])
    lax.fori_loop(0, 128, fill, 0, unroll=4)
    plsc.subcore_barrier()

    def fire(j, _):
        pltpu.async_copy(sbuf, agg_sh.at[idx_v.at[j]], ssem, add=True)
        return 0
    lax.fori_loop(0, PKR, fire, 0)

    def drain(j, _):
        pltpu.make_async_copy(sbuf, agg_sh.at[idx_v.at[0]], ssem).wait()
        return 0
    lax.fori_loop(0, PKR, drain, 0)
    plsc.subcore_barrier()

    pltpu.sync_copy(agg_sh.at[pl.ds(s_ax * RPS, RPS)],
                    p_out.at[c_ax, pl.ds(s_ax * RPS, RPS)])


def _count(pk3, use_src):
    k = pl.kernel(
        functools.partial(_count_body, use_src=use_src),
        out_type=jax.ShapeDtypeStruct((NC, NP, D), jnp.float32),
        mesh=plsc.VectorSubcoreMesh(**_MESH),
        scratch_types=[
            pltpu.VMEM((PKR, 128), jnp.int32),
            pltpu.VMEM((PKR, 128), jnp.int32),
            pltpu.VMEM((128, D), jnp.float32),
            pltpu.VMEM_SHARED((NP, D), jnp.float32),
            pltpu.SemaphoreType.DMA,
        ],
        compiler_params=_SC_PARAMS,
        name="sc_count_src" if use_src else "sc_count_dst",
    )
    return k(pk3)


def _dec_src(j, pk_v, sring):
    row = lax.shift_right_logical(j, 2)
    cb = (j & 3) * CGA
    for k in range(CGA // 16):
        v = pk_v[row, pl.ds(cb + k * 16, 16)]
        sring[j % 8, pl.ds(k * 16, 16)] = lax.shift_right_logical(v, 14)


def _dec_dst(j, pk_v, dring):
    row = lax.shift_right_logical(j, 2)
    cb = (j & 3) * SUB
    for k in range(SUB // 16):
        v = pk_v[row, pl.ds(cb + k * 16, 16)]
        dring[j % 4, pl.ds(k * 16, 16)] = v & 16383


def _spmm_body(h, pk3, a3, p_out, pk_v, a_v, gbuf, sbuf, sring, dring,
               agg_sh, gsem0, gsem1, gsem2, gsem3, ssem0, ssem1):
    c_ax = lax.axis_index("c")
    s_ax = lax.axis_index("s")
    wid = s_ax * NC + c_ax
    pltpu.sync_copy(pk3.at[wid], pk_v)
    pltpu.sync_copy(a3.at[wid], a_v)

    def zfill(i, _):
        for f in range(D // 16):
            gbuf[0, i, pl.ds(f * 16, 16)] = jnp.zeros((16,), jnp.float32)
        return 0
    lax.fori_loop(0, CGA, zfill, 0, unroll=4)
    for z in range(RPS // CGA):
        pltpu.sync_copy(gbuf.at[0],
                        agg_sh.at[pl.ds(s_ax * RPS + z * CGA, CGA)])
    rem = RPS - (RPS // CGA) * CGA
    pltpu.sync_copy(gbuf.at[0, pl.ds(0, rem)],
                    agg_sh.at[pl.ds(s_ax * RPS + (RPS // CGA) * CGA, rem)])

    for b0 in range(4):
        _dec_src(jnp.int32(b0), pk_v, sring)
    gsems_pre = (gsem0, gsem1, gsem2, gsem3)
    for b0 in range(4):
        pltpu.async_copy(h.at[sring.at[b0]], gbuf.at[b0], gsems_pre[b0])
    plsc.subcore_barrier()

    gsems = (gsem0, gsem1, gsem2, gsem3)
    ssems = (ssem0, ssem1)

    def one_chunk(j, b, k):
        sb = b % 2
        pltpu.make_async_copy(h.at[sring.at[j % 8]], gbuf.at[b],
                              gsems[b]).wait()
        _dec_dst(j, pk_v, dring)
        row = lax.shift_right_logical(j, 2)
        cb = (j & 3) * CGA

        def _wait_prev():
            pltpu.make_async_copy(
                sbuf.at[sb], agg_sh.at[dring.at[(j + 2) % 4]],
                ssems[sb]).wait()
        if b >= 2:
            _wait_prev()
        else:
            pl.when(k >= 1)(_wait_prev)
        coefs = [
            plsc.load_gather(
                a_v, [jnp.full((16,), row, jnp.int32),
                      jnp.full((16,), cb + e, jnp.int32)])
            for e in range(CGA)]
        for f in range(D // 16):
            sl = pl.ds(f * 16, 16)
            for e in range(CGA):
                sbuf[sb, e, sl] = gbuf[b, e, sl] * coefs[e]
        pltpu.async_copy(sbuf.at[sb], agg_sh.at[dring.at[j % 4]],
                         ssems[sb], add=True)

        @pl.when(k < NCHUNK // 4 - 1)
        def _():
            _dec_src(j + 4, pk_v, sring)
            pltpu.async_copy(h.at[sring.at[(j + 4) % 8]], gbuf.at[b],
                             gsems[b])

    def body(k, _):
        for b in range(4):
            one_chunk(4 * k + b, b, k)
        return 0
    lax.fori_loop(0, NCHUNK // 4, body, 0)

    for sb in range(2):
        pltpu.make_async_copy(sbuf.at[sb],
                              agg_sh.at[dring.at[(NCHUNK - 2 + sb) % 4]],
                              ssems[sb]).wait()
    plsc.subcore_barrier()

    pltpu.sync_copy(agg_sh.at[pl.ds(s_ax * RPS, RPS)],
                    p_out.at[c_ax, pl.ds(s_ax * RPS, RPS)])


def _spmm(h, pk3, a3):
    k = pl.kernel(
        _spmm_body,
        out_type=jax.ShapeDtypeStruct((NC, NP, D), jnp.float32),
        mesh=plsc.VectorSubcoreMesh(**_MESH),
        scratch_types=[
            pltpu.VMEM((PKR, 128), jnp.int32),
            pltpu.VMEM((PKR, 128), jnp.float32),
            pltpu.VMEM((4, CGA, D), jnp.float32),
            pltpu.VMEM((2, SUB, D), jnp.float32),
            pltpu.VMEM((8, CGA), jnp.int32),
            pltpu.VMEM((4, SUB), jnp.int32),
            pltpu.VMEM_SHARED((NP, D), jnp.float32),
            pltpu.SemaphoreType.DMA,
            pltpu.SemaphoreType.DMA,
            pltpu.SemaphoreType.DMA,
            pltpu.SemaphoreType.DMA,
            pltpu.SemaphoreType.DMA,
            pltpu.SemaphoreType.DMA,
        ],
        compiler_params=_SC_PARAMS,
        name="sc_spmm",
    )
    return k(h, pk3, a3)


def _norms_body(outp_ref, inp_ref, x_ref, norms_ref, xs_ref):
    outd = outp_ref[0, :, 0:DEGW] + outp_ref[1, :, 0:DEGW]
    ind = inp_ref[0, :, 0:DEGW] + inp_ref[1, :, 0:DEGW]
    ns = jnp.where(outd > 0, lax.rsqrt(jnp.maximum(outd, 1.0)), 0.0)
    nd = jnp.where(ind > 0, lax.rsqrt(jnp.maximum(ind, 1.0)), 0.0)
    norms_ref[0] = ns
    norms_ref[1] = nd
    xs = x_ref[...] * ns[:N, 0:1]
    xs_ref[...] = jnp.concatenate(
        [xs, jnp.zeros((NH - N, D), jnp.float32)], axis=0)


def _norms_prescale(outp, inp, x):
    return pl.pallas_call(
        _norms_body,
        out_shape=(
            jax.ShapeDtypeStruct((2, NP, DEGW), jnp.float32),
            jax.ShapeDtypeStruct((NH, D), jnp.float32),
        ),
    )(outp, inp, x)


def _dense_body(p_ref, norms_ref, w_ref, b_ref, g_ref, bt_ref, o_ref, *,
                act, bn, prescale):
    nd = norms_ref[1, :N, 0:1]
    agg = (p_ref[0, :N] + p_ref[1, :N]) * nd
    y = jnp.dot(agg, w_ref[...], preferred_element_type=jnp.float32)
    y = y + b_ref[...][0:1, :]
    if act:
        y = jnp.maximum(y, 0.0)
    if bn:
        mu = jnp.mean(y, axis=0, keepdims=True)
        var = jnp.mean((y - mu) ** 2, axis=0, keepdims=True)
        y = (y - mu) * lax.rsqrt(var + 1e-5) * g_ref[...][0:1, :] \
            + bt_ref[...][0:1, :]
    if prescale:
        y = y * norms_ref[0, :N, 0:1]
        y = jnp.concatenate([y, jnp.zeros((NH - N, D), jnp.float32)], axis=0)
    o_ref[...] = y


def _dense_stage(p, norms, w, b, gamma, beta, act, bn, prescale):
    body = functools.partial(_dense_body, act=act, bn=bn, prescale=prescale)
    rows = NH if prescale else N
    return pl.pallas_call(
        body,
        out_shape=jax.ShapeDtypeStruct((rows, D), jnp.float32),
    )(p, norms, w, b.reshape(1, D), gamma.reshape(1, D), beta.reshape(1, D))


def kernel(x, edge_index, W0, b0, W1, b1, W2, b2, gamma1, beta1):
    ar = jnp.arange(E_PAD - E, dtype=jnp.int32)
    pad_s = JUNK + ar % (NH - N)
    pad_d = JUNK + ar % (NP - N)
    srcp = jnp.concatenate([edge_index[0], pad_s])
    dstp = jnp.concatenate([edge_index[1], pad_d])
    pk3 = ((srcp << 14) | dstp).reshape(NW, PKR, 128)
    nk = jax.random.key(42)
    apad = jnp.zeros((E_PAD - E,), jnp.float32)

    def mk_a(i):
        a = 1.0 + jax.random.normal(jax.random.fold_in(nk, i), (E,),
                                    jnp.float32)
        return jnp.concatenate([a, apad]).reshape(NW, PKR, 128)

    outp = _count(pk3, True)
    inp = _count(pk3, False)
    norms, xs = _norms_prescale(outp, inp, x)

    p = _spmm(xs, pk3, mk_a(0))
    h = _dense_stage(p, norms, W0, b0, gamma1, beta1, True, False, True)
    p = _spmm(h, pk3, mk_a(1))
    h = _dense_stage(p, norms, W1, b1, gamma1, beta1, True, True, True)
    p = _spmm(h, pk3, mk_a(2))
    out = _dense_stage(p, norms, W2, b2, gamma1, beta1, False, False, False)
    return out
